# baseline (speedup 1.0000x reference)
import jax
import jax.numpy as jnp
from jax.experimental import pallas as pl
from jax.experimental.pallas import tpu as pltpu

_EPS2 = 1e-24
_K_OUT = 64
_K_PAD = 80
_B = 4


def _netvlad_body(x_ref, w_ref, b_ref, c_ref, o_ref):
    for i in range(_B):
        _one_image(x_ref.at[i], w_ref, b_ref, c_ref, o_ref.at[i])


def _one_image(x_ref, w_ref, b_ref, c_ref, o_ref):
    x = x_ref[...]

    ssq = jnp.sum(x * x, axis=0, keepdims=True)
    inv = jax.lax.rsqrt(jnp.maximum(ssq, _EPS2))

    xb = x.astype(jnp.bfloat16)

    l0 = jax.lax.dot_general(
        w_ref[...], xb, (((1,), (0,)), ((), ())),
        preferred_element_type=jnp.float32)
    logits = l0 * inv + b_ref[...]

    e = jnp.exp(logits)
    s = jnp.sum(e, axis=0, keepdims=True)
    rcp_s = 1.0 / s

    a = e[:_K_OUT, :] * rcp_s
    asum = jnp.sum(a, axis=1, keepdims=True)

    agg = jax.lax.dot_general(
        (a * inv).astype(jnp.bfloat16), xb, (((1,), (1,)), ((), ())),
        preferred_element_type=jnp.float32)
    vlad = agg - asum * c_ref[...]

    rsq = jnp.sum(vlad * vlad, axis=1, keepdims=True)
    v = vlad * jax.lax.rsqrt(jnp.maximum(rsq, _EPS2))

    gsq = jnp.sum(jnp.sum(v * v, axis=1, keepdims=True),
                  axis=0, keepdims=True)
    o_ref[...] = v * jax.lax.rsqrt(jnp.maximum(gsq, _EPS2))


def kernel(x, conv_w, conv_b, centroids):
    N, C, H, W = x.shape
    K_all = conv_w.shape[0]
    P = H * W

    xf = x.reshape(N, C, P)
    pad = _K_PAD - K_all
    w_p = jnp.pad(conv_w, ((0, pad), (0, 0))).astype(jnp.bfloat16)
    b_p = jnp.pad(conv_b, ((0, pad),), constant_values=-1e30).reshape(_K_PAD, 1)
    c_k = centroids[:_K_OUT]

    out = pl.pallas_call(
        _netvlad_body,
        grid=(N // _B,),
        in_specs=[
            pl.BlockSpec((_B, C, P), lambda n: (n, 0, 0)),
            pl.BlockSpec((_K_PAD, C), lambda n: (0, 0)),
            pl.BlockSpec((_K_PAD, 1), lambda n: (0, 0)),
            pl.BlockSpec((_K_OUT, C), lambda n: (0, 0)),
        ],
        out_specs=pl.BlockSpec((_B, _K_OUT, C), lambda n: (n, 0, 0)),
        out_shape=jax.ShapeDtypeStruct((N, _K_OUT, C), jnp.float32),
        compiler_params=pltpu.CompilerParams(
            dimension_semantics=("parallel",),
            vmem_limit_bytes=56 * 1024 * 1024,
        ),
    )(xf, w_p, b_p, c_k)

    return out.reshape(N, _K_OUT * C)
